# Initial kernel scaffold; baseline (speedup 1.0000x reference)
import jax
import jax.numpy as jnp
from jax.experimental import pallas as pl


def kernel(distance, n_residues):
    raise NotImplementedError("write your pallas kernel here")


# speedup vs baseline: 5.3732x; 5.3732x over previous
import functools
import math

import jax
import jax.numpy as jnp
from jax import lax
from jax.experimental import pallas as pl
from jax.experimental.pallas import tpu as pltpu
from jax.experimental.pallas import tpu_sc as plsc

_SCALE = 1000.0


def _tc_max_body(x_ref, o_ref):
    i = pl.program_id(0)
    bm = jnp.max(x_ref[...])

    @pl.when(i == 0)
    def _():
        o_ref[...] = jnp.full((1, 128), bm, jnp.float32)

    @pl.when(i != 0)
    def _():
        o_ref[...] = jnp.maximum(o_ref[...], bm)


def _global_max_128(x2d, rows_per_block):
    rows = x2d.shape[0]
    return pl.pallas_call(
        _tc_max_body,
        grid=(rows // rows_per_block,),
        in_specs=[pl.BlockSpec((rows_per_block, 128), lambda i: (i, 0))],
        out_specs=pl.BlockSpec((1, 128), lambda i: (0, 0)),
        out_shape=jax.ShapeDtypeStruct((1, 128), jnp.float32),
    )(x2d)


@functools.lru_cache(maxsize=None)
def _make_sc_penalty(n, size):
    info = plsc.get_sparse_core_info()
    ns = info.num_subcores
    nw = ns
    nt = n - 2
    tpw = ((nt + nw - 1) // nw + 15) // 16 * 16
    k_chunks = tpw // 16
    mesh = plsc.VectorSubcoreMesh(
        core_axis_name="c", subcore_axis_name="s", num_cores=1)

    @functools.partial(
        pl.kernel,
        mesh=mesh,
        compiler_params=pltpu.CompilerParams(needs_layout_passes=False),
        out_type=jax.ShapeDtypeStruct((16,), jnp.float32),
        scratch_types=[
            pltpu.VMEM((tpw,), jnp.int32),
            pltpu.VMEM((tpw,), jnp.int32),
            pltpu.VMEM((tpw,), jnp.int32),
            pltpu.VMEM((tpw,), jnp.float32),
            pltpu.VMEM((tpw,), jnp.float32),
            pltpu.VMEM((tpw,), jnp.float32),
            pltpu.VMEM((16,), jnp.float32),
            pltpu.VMEM((16,), jnp.float32),
            pltpu.VMEM((nw * 16,), jnp.float32),
            pltpu.VMEM_SHARED((nw * 16,), jnp.float32),
            pltpu.SemaphoreType.DMA,
            pltpu.SemaphoreType.DMA,
            pltpu.SemaphoreType.DMA,
        ],
    )
    def sc_penalty(dist_hbm, m_hbm, out_hbm, idx1, idx2, idx3, v1, v2, v3,
                   mv, pv, pall, shared, s1, s2, s3):
        wid = lax.axis_index("s")
        base = wid * tpw
        iota = lax.iota(jnp.int32, 16)
        for k in range(k_chunks):
            i_vec = base + (16 * k) + iota
            ia = jnp.minimum(i_vec, nt)
            ib = jnp.minimum(i_vec + 1, nt)
            a1 = (n + 1) * ia - lax.shift_right_logical(ia * (ia + 1), 1) + 1
            a2 = (n + 1) * ib - lax.shift_right_logical(ib * (ib + 1), 1) + 1
            idx1[pl.ds(16 * k, 16)] = a1
            idx2[pl.ds(16 * k, 16)] = a2
            idx3[pl.ds(16 * k, 16)] = a1 + 1
        c1 = pltpu.async_copy(dist_hbm.at[idx1], v1, s1)
        c2 = pltpu.async_copy(dist_hbm.at[idx2], v2, s2)
        c3 = pltpu.async_copy(dist_hbm.at[idx3], v3, s3)
        c1.wait()
        c2.wait()
        c3.wait()
        acc = jnp.zeros((16,), jnp.float32)
        for k in range(k_chunks):
            d1 = jnp.exp(v1[pl.ds(16 * k, 16)])
            d2 = jnp.exp(v2[pl.ds(16 * k, 16)])
            d3 = jnp.exp(v3[pl.ds(16 * k, 16)])
            mx = jnp.maximum(d1, jnp.maximum(d2, d3))
            pen = jnp.maximum(2.0 * mx - (d1 + d2 + d3), 0.0)
            t_vec = base + (16 * k) + iota
            acc = acc + jnp.where(t_vec < nt, pen, 0.0)
        pv[...] = acc
        pltpu.sync_copy(pv, shared.at[pl.ds(wid * 16, 16)])
        plsc.subcore_barrier()

        @pl.when(wid == 0)
        def _():
            pltpu.sync_copy(m_hbm.at[pl.ds(0, 16)], mv)
            pltpu.sync_copy(shared, pall)
            tot = pall[pl.ds(0, 16)]
            for w in range(1, nw):
                tot = tot + pall[pl.ds(16 * w, 16)]
            s = jnp.sum(tot)
            pv[...] = (s * _SCALE) / jnp.exp(mv[...])
            pltpu.sync_copy(pv, out_hbm)

    return sc_penalty


def kernel(distance, n_residues):
    size = distance.shape[0]
    n = (math.isqrt(8 * size + 1) - 1) // 2
    rows = size // 128
    rows_per_block = rows // 3 if (rows % 3 == 0 and (rows // 3) % 8 == 0) else rows
    m128 = _global_max_128(distance.reshape(rows, 128), rows_per_block)
    out16 = _make_sc_penalty(n, size)(distance, m128.reshape(128))
    return out16[0]


# speedup vs baseline: 5.5739x; 1.0374x over previous
import functools
import math

import jax
import jax.numpy as jnp
from jax import lax
from jax.experimental import pallas as pl
from jax.experimental.pallas import tpu as pltpu
from jax.experimental.pallas import tpu_sc as plsc

_SCALE = 1000.0


def _tc_max_body(x_ref, o_ref):
    i = pl.program_id(0)
    bm = jnp.max(x_ref[...])

    @pl.when(i == 0)
    def _():
        o_ref[...] = jnp.full((1, 128), bm, jnp.float32)

    @pl.when(i != 0)
    def _():
        o_ref[...] = jnp.maximum(o_ref[...], bm)


def _global_max_128(x2d, rows_per_block):
    rows = x2d.shape[0]
    return pl.pallas_call(
        _tc_max_body,
        grid=(rows // rows_per_block,),
        in_specs=[pl.BlockSpec((rows_per_block, 128), lambda i: (i, 0))],
        out_specs=pl.BlockSpec((1, 128), lambda i: (0, 0)),
        out_shape=jax.ShapeDtypeStruct((1, 128), jnp.float32),
    )(x2d)


@functools.lru_cache(maxsize=None)
def _make_sc_penalty(n, size):
    info = plsc.get_sparse_core_info()
    ns = info.num_subcores
    nw = ns
    nt = n - 2
    tpw = ((nt + nw - 1) // nw + 15) // 16 * 16
    k_chunks = tpw // 16
    mesh = plsc.VectorSubcoreMesh(
        core_axis_name="c", subcore_axis_name="s", num_cores=1)

    @functools.partial(
        pl.kernel,
        mesh=mesh,
        compiler_params=pltpu.CompilerParams(needs_layout_passes=False),
        out_type=jax.ShapeDtypeStruct((16,), jnp.float32),
        scratch_types=[
            pltpu.VMEM((tpw,), jnp.int32),
            pltpu.VMEM((tpw,), jnp.int32),
            pltpu.VMEM((tpw,), jnp.int32),
            pltpu.VMEM((tpw,), jnp.float32),
            pltpu.VMEM((tpw,), jnp.float32),
            pltpu.VMEM((tpw,), jnp.float32),
            pltpu.VMEM((16,), jnp.float32),
            pltpu.VMEM((nw * 16,), jnp.float32),
            pltpu.VMEM_SHARED((nw * 16,), jnp.float32),
            pltpu.SemaphoreType.DMA,
            pltpu.SemaphoreType.DMA,
            pltpu.SemaphoreType.DMA,
        ],
    )
    def sc_penalty(dist_hbm, out_hbm, idx1, idx2, idx3, v1, v2, v3,
                   pv, pall, shared, s1, s2, s3):
        wid = lax.axis_index("s")
        base = wid * tpw
        iota = lax.iota(jnp.int32, 16)
        for k in range(k_chunks):
            i_vec = base + (16 * k) + iota
            ia = jnp.minimum(i_vec, nt)
            ib = jnp.minimum(i_vec + 1, nt)
            a1 = (n + 1) * ia - lax.shift_right_logical(ia * (ia + 1), 1) + 1
            a2 = (n + 1) * ib - lax.shift_right_logical(ib * (ib + 1), 1) + 1
            idx1[pl.ds(16 * k, 16)] = a1
            idx2[pl.ds(16 * k, 16)] = a2
            idx3[pl.ds(16 * k, 16)] = a1 + 1
        c1 = pltpu.async_copy(dist_hbm.at[idx1], v1, s1)
        c2 = pltpu.async_copy(dist_hbm.at[idx2], v2, s2)
        c3 = pltpu.async_copy(dist_hbm.at[idx3], v3, s3)
        c1.wait()
        c2.wait()
        c3.wait()
        acc = jnp.zeros((16,), jnp.float32)
        for k in range(k_chunks):
            d1 = jnp.exp(v1[pl.ds(16 * k, 16)])
            d2 = jnp.exp(v2[pl.ds(16 * k, 16)])
            d3 = jnp.exp(v3[pl.ds(16 * k, 16)])
            mx = jnp.maximum(d1, jnp.maximum(d2, d3))
            pen = jnp.maximum(2.0 * mx - (d1 + d2 + d3), 0.0)
            t_vec = base + (16 * k) + iota
            acc = acc + jnp.where(t_vec < nt, pen, 0.0)
        pv[...] = acc
        pltpu.sync_copy(pv, shared.at[pl.ds(wid * 16, 16)])
        plsc.subcore_barrier()

        @pl.when(wid == 0)
        def _():
            pltpu.sync_copy(shared, pall)
            tot = pall[pl.ds(0, 16)]
            for w in range(1, nw):
                tot = tot + pall[pl.ds(16 * w, 16)]
            s = jnp.sum(tot)
            pv[...] = jnp.zeros((16,), jnp.float32) + s
            pltpu.sync_copy(pv, out_hbm)

    return sc_penalty


def kernel(distance, n_residues):
    size = distance.shape[0]
    n = (math.isqrt(8 * size + 1) - 1) // 2
    rows = size // 128
    rows_per_block = rows // 3 if (rows % 3 == 0 and (rows // 3) % 8 == 0) else rows
    pen16 = _make_sc_penalty(n, size)(distance)
    m128 = _global_max_128(distance.reshape(rows, 128), rows_per_block)
    return pen16[0] * _SCALE / jnp.exp(m128[0, 0])


# speedup vs baseline: 5.5852x; 1.0020x over previous
import functools
import math

import jax
import jax.numpy as jnp
from jax import lax
from jax.experimental import pallas as pl
from jax.experimental.pallas import tpu as pltpu
from jax.experimental.pallas import tpu_sc as plsc

_SCALE = 1000.0


def _tc_max_body(x_ref, o_ref):
    i = pl.program_id(0)
    bm = jnp.max(x_ref[...])

    @pl.when(i == 0)
    def _():
        o_ref[...] = jnp.full((1, 128), bm, jnp.float32)

    @pl.when(i != 0)
    def _():
        o_ref[...] = jnp.maximum(o_ref[...], bm)


def _global_max_128(x2d, rows_per_block):
    rows = x2d.shape[0]
    return pl.pallas_call(
        _tc_max_body,
        grid=(rows // rows_per_block,),
        in_specs=[pl.BlockSpec((rows_per_block, 128), lambda i: (i, 0))],
        out_specs=pl.BlockSpec((1, 128), lambda i: (0, 0)),
        out_shape=jax.ShapeDtypeStruct((1, 128), jnp.float32),
    )(x2d)


@functools.lru_cache(maxsize=None)
def _make_sc_penalty(n, size):
    info = plsc.get_sparse_core_info()
    ns = info.num_subcores
    nw = ns
    nt = n - 2
    tpw = ((nt + nw - 1) // nw + 15) // 16 * 16
    k_chunks = tpw // 16
    amax = size - 2
    mesh = plsc.VectorSubcoreMesh(
        core_axis_name="c", subcore_axis_name="s", num_cores=1)

    @functools.partial(
        pl.kernel,
        mesh=mesh,
        compiler_params=pltpu.CompilerParams(needs_layout_passes=False),
        out_type=jax.ShapeDtypeStruct((16,), jnp.float32),
        scratch_types=[
            pltpu.VMEM((tpw,), jnp.int32),
            pltpu.VMEM((tpw,), jnp.int32),
            pltpu.VMEM((tpw,), jnp.int32),
            pltpu.VMEM((tpw,), jnp.float32),
            pltpu.VMEM((tpw,), jnp.float32),
            pltpu.VMEM((tpw,), jnp.float32),
            pltpu.VMEM((16,), jnp.float32),
            pltpu.VMEM((nw * 16,), jnp.float32),
            pltpu.VMEM_SHARED((nw * 16,), jnp.float32),
            pltpu.SemaphoreType.DMA,
            pltpu.SemaphoreType.DMA,
            pltpu.SemaphoreType.DMA,
        ],
    )
    def sc_penalty(dist_hbm, out_hbm, idx1, idx2, idx3, v1, v2, v3,
                   pv, pall, shared, s1, s2, s3):
        wid = lax.axis_index("s")
        base = wid * tpw
        iota = lax.iota(jnp.int32, 16)
        i0 = base + iota
        a0 = (n + 1) * i0 - lax.shift_right_logical(i0 * (i0 + 1), 1) + 1

        def idx_body(k, carry):
            i_vec, a1 = carry
            a2 = a1 + (n - i_vec)
            idx1[pl.ds(16 * k, 16)] = jnp.minimum(a1, amax)
            idx2[pl.ds(16 * k, 16)] = jnp.minimum(a2, amax)
            idx3[pl.ds(16 * k, 16)] = jnp.minimum(a1, amax) + 1
            return (i_vec + 16, a1 + 16 * (n - i_vec) - 120)

        lax.fori_loop(0, k_chunks, idx_body, (i0, a0), unroll=False)
        c1 = pltpu.async_copy(dist_hbm.at[idx1], v1, s1)
        c2 = pltpu.async_copy(dist_hbm.at[idx2], v2, s2)
        c3 = pltpu.async_copy(dist_hbm.at[idx3], v3, s3)
        c1.wait()
        c2.wait()
        c3.wait()

        def pen_body(k, acc):
            d1 = jnp.exp(v1[pl.ds(16 * k, 16)])
            d2 = jnp.exp(v2[pl.ds(16 * k, 16)])
            d3 = jnp.exp(v3[pl.ds(16 * k, 16)])
            mx = jnp.maximum(d1, jnp.maximum(d2, d3))
            pen = jnp.maximum(2.0 * mx - (d1 + d2 + d3), 0.0)
            t_vec = base + 16 * k + iota
            return acc + jnp.where(t_vec < nt, pen, 0.0)

        acc = lax.fori_loop(0, k_chunks, pen_body,
                            jnp.zeros((16,), jnp.float32), unroll=False)
        pv[...] = acc
        pltpu.sync_copy(pv, shared.at[pl.ds(wid * 16, 16)])
        plsc.subcore_barrier()

        @pl.when(wid == 0)
        def _():
            pltpu.sync_copy(shared, pall)

            def red_body(w, tot):
                return tot + pall[pl.ds(16 * w, 16)]

            tot = lax.fori_loop(1, nw, red_body, pall[pl.ds(0, 16)],
                                unroll=False)
            s = jnp.sum(tot)
            pv[...] = jnp.zeros((16,), jnp.float32) + s
            pltpu.sync_copy(pv, out_hbm)

    return sc_penalty


def kernel(distance, n_residues):
    size = distance.shape[0]
    n = (math.isqrt(8 * size + 1) - 1) // 2
    rows = size // 128
    rows_per_block = rows // 3 if (rows % 3 == 0 and (rows // 3) % 8 == 0) else rows
    pen16 = _make_sc_penalty(n, size)(distance)
    m128 = _global_max_128(distance.reshape(rows, 128), rows_per_block)
    return pen16[0] * _SCALE / jnp.exp(m128[0, 0])
